# baseline (speedup 1.0000x reference)
import functools

import jax
import jax.numpy as jnp
from jax import lax
from jax.experimental import pallas as pl
from jax.experimental.pallas import tpu as pltpu
from jax.experimental.pallas import tpu_sc as plsc

N = 50000
E = 800000
D_IN = 100
H = 64
C = 18

NC = 2
NS = 16
LANES = 16

HALF = N // 2
ACC_ROWS = 25088
PAD = ACC_ROWS - HALF
N_P = 2 * ACC_ROWS
TRASH = HALF
STRIPE = ACC_ROWS // NS


CH = 128
SUB = 128
NSUB = CH // SUB
GPS = SUB // LANES
NCHUNK = 394
PT = NCHUNK * CH
E_P = NS * PT
MROWS = E_P // SUB

CNT_STR = 1584
CNT_DOM = NS * CNT_STR
CNT_P = 2 * CNT_DOM
CNT_CH = 2000
CNT_NCH = (E // NS) // CNT_CH


def _make_seg(weighted: bool):
    mesh = plsc.VectorSubcoreMesh(core_axis_name="c", subcore_axis_name="s")
    scratch = (
        [pltpu.VMEM((3, SUB), jnp.int32)] * 2
        + [pltpu.VMEM((NSUB, SUB), jnp.int32)] * 2
        + [pltpu.VMEM((NSUB, SUB), jnp.int32)] * 2
        + [pltpu.VMEM((CH, H), jnp.float32)] * 2
        + [pltpu.SemaphoreType.DMA] * 2
        + [pltpu.SemaphoreType.DMA] * 2
        + [pltpu.SemaphoreType.DMA] * 2
        + [pltpu.VMEM_SHARED((ACC_ROWS, H), jnp.float32)]
    )

    @functools.partial(
        pl.kernel,
        out_type=jax.ShapeDtypeStruct((N_P, H), jnp.float32),
        mesh=mesh,
        scratch_types=scratch,
        compiler_params=pltpu.CompilerParams(
            needs_layout_passes=False, use_tc_tiling_on_sc=False),
    )
    def seg(h_hbm, meta_hbm, out_hbm,
            meta0, meta1, sidx0, sidx1, scat0, scat1, rows0, rows1,
            msem0, msem1, gsem0, gsem1, ssem0, ssem1, acc):
        cid = lax.axis_index("c")
        sid = lax.axis_index("s")
        base = cid * HALF
        metas = (meta0, meta1)
        sidxs = (sidx0, sidx1)
        scats = (scat0, scat1)
        rowss = (rows0, rows1)
        msems = (msem0, msem1)
        gsems = (gsem0, gsem1)
        ssems = (ssem0, ssem1)

        @pl.loop(0, CH)
        def _(r):
            for f in range(H // LANES):
                rows0[r, pl.ds(f * LANES, LANES)] = jnp.zeros((LANES,), jnp.float32)
        s0 = sid * STRIPE
        for kz in range(STRIPE // CH):
            pltpu.sync_copy(rows0, acc.at[pl.ds(s0 + kz * CH, CH)])
        rem = STRIPE % CH
        if rem:
            pltpu.sync_copy(rows0.at[pl.ds(0, rem)],
                            acc.at[pl.ds(s0 + (STRIPE // CH) * CH, rem)])
        plsc.subcore_barrier()

        tile_r0 = sid * (PT // SUB)

        def drain_rows_bytes(sem, b):
            pltpu.make_async_copy(h_hbm.at[pl.ds(0, SUB), :], rowss[b],
                                  sem).wait()

        def meta_slice(ci):
            return meta_hbm.at[tile_r0 + ci]

        def build_idx(b):
            meta, sidx = metas[b], sidxs[b]
            for g in range(CH // LANES):
                k = g * LANES
                sv = meta[0, pl.ds(k, LANES)]
                meta[0, pl.ds(k, LANES)] = jnp.where(sv >= HALF, sv + PAD, sv)
                dv = meta[1, pl.ds(k, LANES)] - base
                ok = (dv >= 0) & (dv < HALF)
                sidx[0, pl.ds(k, LANES)] = jnp.where(ok, dv, TRASH)

        def issue_gather(b):
            pltpu.async_copy(h_hbm.at[metas[b].at[0]], rowss[b], gsems[b])

        def multiply(b):
            meta, rows = metas[b], rowss[b]

            @pl.loop(0, GPS)
            def _(gg):
                wv = plsc.bitcast(
                    meta[2, pl.ds(gg * LANES, LANES)], jnp.float32)
                e0g = gg * LANES
                for l in range(LANES):
                    w = wv[l]
                    for f in range(H // LANES):
                        rows[e0g + l, pl.ds(f * LANES, LANES)] = (
                            rows[e0g + l, pl.ds(f * LANES, LANES)] * w)

        def issue_scatter(b):
            sidx, scat, rows = sidxs[b], scats[b], rowss[b]
            for g in range(CH // LANES):
                k = g * LANES
                scat[0, pl.ds(k, LANES)] = sidx[0, pl.ds(k, LANES)]
            pltpu.async_copy(rows, acc.at[scat.at[0]], ssems[b], add=True)

        def slot(b, nxt, has_next, has_prev):
            nb = 1 - b
            if has_next:
                pltpu.async_copy(meta_slice(nxt), metas[nb], msems[nb])
            drain_rows_bytes(gsems[b], b)
            if weighted:
                multiply(b)
            if has_next:
                pltpu.make_async_copy(meta_slice(0), metas[nb],
                                      msems[nb]).wait()
                build_idx(nb)
            if has_prev:
                drain_rows_bytes(ssems[nb], nb)
            if has_next:
                issue_gather(nb)
            issue_scatter(b)

        pltpu.sync_copy(meta_slice(0), meta0)
        build_idx(0)
        issue_gather(0)

        slot(0, 1, True, False)

        @pl.loop(1, NCHUNK - 2, step=2)
        def _(ci):
            slot(1, ci + 1, True, True)
            slot(0, ci + 2, True, True)

        slot(1, 0, False, True)
        drain_rows_bytes(ssems[1], 1)
        plsc.subcore_barrier()
        pltpu.sync_copy(
            acc.at[pl.ds(sid * STRIPE, STRIPE)],
            out_hbm.at[pl.ds(cid * ACC_ROWS + sid * STRIPE, STRIPE)])

    return seg


_seg_weighted = _make_seg(True)
_seg_unweighted = _make_seg(False)


def _make_cnt():
    mesh = plsc.VectorSubcoreMesh(core_axis_name="c", subcore_axis_name="s")
    scratch = [
        pltpu.VMEM((CNT_CH,), jnp.int32),
        pltpu.VMEM((CNT_CH,), jnp.int32),
        pltpu.VMEM((CNT_DOM,), jnp.float32),
        pltpu.VMEM((CNT_STR,), jnp.float32),
        pltpu.VMEM((CNT_STR,), jnp.float32),
        pltpu.SemaphoreType.DMA,
        pltpu.SemaphoreType.DMA,
        pltpu.VMEM_SHARED((NS, CNT_DOM), jnp.float32),
    ]

    @functools.partial(
        pl.kernel,
        out_type=jax.ShapeDtypeStruct((CNT_P,), jnp.float32),
        mesh=mesh,
        scratch_types=scratch,
        compiler_params=pltpu.CompilerParams(
            needs_layout_passes=False, use_tc_tiling_on_sc=False),
    )
    def cnt(dst_hbm, out_hbm, dstb0, dstb1, hist, tmp, tmp2,
            csem0, csem1, shared):
        cid = lax.axis_index("c")
        sid = lax.axis_index("s")
        base = cid * HALF
        tile_e0 = sid * (E // NS)
        dstbs = (dstb0, dstb1)
        csems = (csem0, csem1)

        @pl.loop(0, CNT_DOM // LANES)
        def _(r):
            hist[pl.ds(r * LANES, LANES)] = jnp.zeros((LANES,), jnp.float32)

        ones = jnp.ones((LANES,), jnp.float32)

        def issue_load(ci, b):
            pltpu.async_copy(dst_hbm.at[pl.ds(tile_e0 + ci * CNT_CH, CNT_CH)],
                             dstbs[b], csems[b])

        def wait_load(b):
            pltpu.make_async_copy(dst_hbm.at[pl.ds(0, CNT_CH)], dstbs[b],
                                  csems[b]).wait()

        def scan(b):
            dstb = dstbs[b]

            @pl.loop(0, CNT_CH // LANES)
            def _(g):
                dv = dstb[pl.ds(g * LANES, LANES)] - base
                ok = (dv >= 0) & (dv < HALF)
                adj = jnp.where(ok, dv, TRASH)
                plsc.addupdate_scatter(hist, [adj], ones)

        issue_load(0, 0)
        issue_load(1, 1)
        wait_load(0)
        scan(0)
        issue_load(2, 0)

        @pl.loop(1, CNT_NCH - 1, step=2)
        def _(ci):
            wait_load(1)
            scan(1)
            issue_load(ci + 2, 1)
            wait_load(0)
            scan(0)
            issue_load(ci + 3, 0)

        wait_load(1)
        wait_load(0)

        pltpu.sync_copy(hist, shared.at[sid])
        plsc.subcore_barrier()

        s0 = sid * CNT_STR
        pltpu.sync_copy(shared.at[0, pl.ds(s0, CNT_STR)], tmp)

        @pl.loop(1, NS)
        def _(t):
            pltpu.sync_copy(shared.at[t, pl.ds(s0, CNT_STR)], tmp2)

            @pl.loop(0, CNT_STR // LANES)
            def _(r):
                tmp[pl.ds(r * LANES, LANES)] = (
                    tmp[pl.ds(r * LANES, LANES)]
                    + tmp2[pl.ds(r * LANES, LANES)])

        @pl.loop(0, CNT_STR // LANES)
        def _(r):
            v = tmp[pl.ds(r * LANES, LANES)]
            tmp[pl.ds(r * LANES, LANES)] = 1.0 / jnp.maximum(v, 1.0)
        pltpu.sync_copy(tmp, out_hbm.at[pl.ds(cid * CNT_DOM + s0, CNT_STR)])

    return cnt


_cnt_kernel = _make_cnt()


_TC_BLK = 128
_TC_GRID = N_P // _TC_BLK


def _emb_body(x_ref, w_ref, b_ref, o_ref):
    o_ref[...] = jnp.maximum(
        jnp.dot(x_ref[...], w_ref[...], preferred_element_type=jnp.float32)
        + b_ref[...], 0.0)


def _emb_tc(x_p, w, b):
    return pl.pallas_call(
        _emb_body,
        grid=(_TC_GRID,),
        in_specs=[
            pl.BlockSpec((_TC_BLK, D_IN), lambda i: (i, 0)),
            pl.BlockSpec((D_IN, H), lambda i: (0, 0)),
            pl.BlockSpec((1, H), lambda i: (0, 0)),
        ],
        out_specs=pl.BlockSpec((_TC_BLK, H), lambda i: (i, 0)),
        out_shape=jax.ShapeDtypeStruct((N_P, H), jnp.float32),
    )(x_p, w, b)


def _layer_body(relu, s_ref, inv_ref, h_ref, wl_ref, bl_ref, wr_ref, o_ref):
    agg = s_ref[...] * inv_ref[...]
    o = (jnp.dot(agg, wl_ref[...], preferred_element_type=jnp.float32)
         + jnp.dot(h_ref[...], wr_ref[...], preferred_element_type=jnp.float32)
         + bl_ref[...])
    nrm = jnp.sqrt(jnp.sum(o * o, axis=-1, keepdims=True))
    o = o / jnp.maximum(nrm, 1e-12)
    if relu:
        o = jnp.maximum(o, 0.0)
    o_ref[...] = o


def _make_layer_tc(relu, width):
    def run(s_p, inv_p, h_p, wl, bl, wr):
        return pl.pallas_call(
            functools.partial(_layer_body, relu),
            grid=(_TC_GRID,),
            in_specs=[
                pl.BlockSpec((_TC_BLK, H), lambda i: (i, 0)),
                pl.BlockSpec((_TC_BLK, 1), lambda i: (i, 0)),
                pl.BlockSpec((_TC_BLK, H), lambda i: (i, 0)),
                pl.BlockSpec((H, width), lambda i: (0, 0)),
                pl.BlockSpec((1, width), lambda i: (0, 0)),
                pl.BlockSpec((H, width), lambda i: (0, 0)),
            ],
            out_specs=pl.BlockSpec((_TC_BLK, width), lambda i: (i, 0)),
            out_shape=jax.ShapeDtypeStruct((N_P, width), jnp.float32),
        )(s_p, inv_p, h_p, wl, bl, wr)
    return run


_layer_hidden = _make_layer_tc(True, H)
_layer_out = _make_layer_tc(False, C)


def kernel(x, edge_index, edge_weight, W_emb, b_emb, Wl0, bl0, Wr0,
           Wl1, bl1, Wr1, Wl2, bl2, Wr2, Wlo, blo, Wro):
    src = edge_index[0]
    dst = edge_index[1]
    zpad = jnp.zeros((PAD, D_IN), jnp.float32)
    x_p = jnp.concatenate([x[:HALF], zpad, x[HALF:], zpad], axis=0)

    epad = E_P - E
    src_p = jnp.concatenate([src, jnp.zeros((epad,), jnp.int32)])
    dst_p = jnp.concatenate([dst, jnp.full((epad,), N, jnp.int32)])
    ew_p = jnp.concatenate([lax.bitcast_convert_type(edge_weight, jnp.int32),
                            jnp.zeros((epad,), jnp.int32)])
    meta = jnp.stack([src_p.reshape(MROWS, SUB), dst_p.reshape(MROWS, SUB),
                      ew_p.reshape(MROWS, SUB)], axis=1)

    h = _emb_tc(x_p, W_emb, b_emb.reshape(1, H))
    cnt_raw = _cnt_kernel(dst_p)
    inv_p = jnp.concatenate(
        [cnt_raw[:ACC_ROWS],
         cnt_raw[CNT_DOM:CNT_DOM + ACC_ROWS]]).reshape(N_P, 1)

    for Wl, bl, Wr in ((Wl0, bl0, Wr0), (Wl1, bl1, Wr1), (Wl2, bl2, Wr2)):
        s = _seg_weighted(h, meta)
        h = _layer_hidden(s, inv_p, h, Wl, bl.reshape(1, H), Wr)

    s = _seg_unweighted(h, meta)
    o_p = _layer_out(s, inv_p, h, Wlo, blo.reshape(1, C), Wro)
    return jnp.concatenate([o_p[:HALF], o_p[ACC_ROWS:ACC_ROWS + HALF]], axis=0)


# speedup vs baseline: 1.4348x; 1.4348x over previous
import functools

import jax
import jax.numpy as jnp
from jax import lax
from jax.experimental import pallas as pl
from jax.experimental.pallas import tpu as pltpu
from jax.experimental.pallas import tpu_sc as plsc

N = 50000
E = 800000
D_IN = 100
H = 64
C = 18

NC = 2
NS = 16
LANES = 16

HALF = N // 2
ACC_ROWS = 25088
PAD = ACC_ROWS - HALF
N_P = 2 * ACC_ROWS
TRASH = HALF
STRIPE = ACC_ROWS // NS


CH = 192
SUB = 96
NSUB = CH // SUB
GPS = SUB // LANES
NCHUNK = 262
PT = NCHUNK * CH
E_P = NS * PT
MROWS = E_P // SUB

CNT_STR = 1584
CNT_DOM = NS * CNT_STR
CNT_P = 2 * CNT_DOM
CNT_CH = 2000
CNT_NCH = (E // NS) // CNT_CH


def _make_seg(weighted: bool):
    mesh = plsc.VectorSubcoreMesh(core_axis_name="c", subcore_axis_name="s")
    scratch = (
        [pltpu.VMEM((NSUB, 3, SUB), jnp.int32)] * 2
        + [pltpu.VMEM((NSUB, SUB), jnp.int32)] * 2
        + [pltpu.VMEM((NSUB, SUB), jnp.int32)] * 2
        + [pltpu.VMEM((CH, H), jnp.float32)] * 2
        + [pltpu.SemaphoreType.DMA] * 2
        + [pltpu.SemaphoreType.DMA] * 2
        + [pltpu.SemaphoreType.DMA] * 2
        + [pltpu.VMEM_SHARED((ACC_ROWS, H), jnp.float32)]
    )

    @functools.partial(
        pl.kernel,
        out_type=jax.ShapeDtypeStruct((N_P, H), jnp.float32),
        mesh=mesh,
        scratch_types=scratch,
        compiler_params=pltpu.CompilerParams(
            needs_layout_passes=False, use_tc_tiling_on_sc=False),
    )
    def seg(h_hbm, meta_hbm, out_hbm,
            meta0, meta1, sidx0, sidx1, scat0, scat1, rows0, rows1,
            msem0, msem1, gsem0, gsem1, ssem0, ssem1, acc):
        cid = lax.axis_index("c")
        sid = lax.axis_index("s")
        base = cid * HALF
        metas = (meta0, meta1)
        sidxs = (sidx0, sidx1)
        scats = (scat0, scat1)
        rowss = (rows0, rows1)
        msems = (msem0, msem1)
        gsems = (gsem0, gsem1)
        ssems = (ssem0, ssem1)

        @pl.loop(0, CH)
        def _(r):
            for f in range(H // LANES):
                rows0[r, pl.ds(f * LANES, LANES)] = jnp.zeros((LANES,), jnp.float32)
        s0 = sid * STRIPE
        for kz in range(STRIPE // CH):
            pltpu.sync_copy(rows0, acc.at[pl.ds(s0 + kz * CH, CH)])
        rem = STRIPE % CH
        if rem:
            pltpu.sync_copy(rows0.at[pl.ds(0, rem)],
                            acc.at[pl.ds(s0 + (STRIPE // CH) * CH, rem)])
        plsc.subcore_barrier()

        tile_r0 = sid * (PT // SUB)

        def drain_rows_bytes(sem, b):
            for j in range(NSUB):
                pltpu.make_async_copy(h_hbm.at[pl.ds(0, SUB), :],
                                      rowss[b].at[pl.ds(j * SUB, SUB)],
                                      sem).wait()

        def meta_slice(ci):
            return meta_hbm.at[pl.ds(tile_r0 + ci * NSUB, NSUB), :, :]

        def build_idx(b):
            meta, sidx = metas[b], sidxs[b]
            for g in range(CH // LANES):
                j, k = g // GPS, (g % GPS) * LANES
                sv = meta[j, 0, pl.ds(k, LANES)]
                meta[j, 0, pl.ds(k, LANES)] = jnp.where(sv >= HALF, sv + PAD, sv)
                dv = meta[j, 1, pl.ds(k, LANES)] - base
                ok = (dv >= 0) & (dv < HALF)
                sidx[j, pl.ds(k, LANES)] = jnp.where(ok, dv, TRASH)

        def issue_gather(b):
            for j in range(NSUB):
                pltpu.async_copy(h_hbm.at[metas[b].at[j, 0]],
                                 rowss[b].at[pl.ds(j * SUB, SUB)], gsems[b])

        def multiply(b):
            meta, rows = metas[b], rowss[b]

            @pl.loop(0, NSUB)
            def _(j):
                @pl.loop(0, GPS)
                def _(gg):
                    wv = plsc.bitcast(
                        meta[j, 2, pl.ds(gg * LANES, LANES)], jnp.float32)
                    e0g = j * SUB + gg * LANES
                    for l in range(LANES):
                        w = wv[l]
                        for f in range(H // LANES):
                            rows[e0g + l, pl.ds(f * LANES, LANES)] = (
                                rows[e0g + l, pl.ds(f * LANES, LANES)] * w)

        def issue_scatter(b):
            sidx, scat, rows = sidxs[b], scats[b], rowss[b]
            for g in range(CH // LANES):
                j, k = g // GPS, (g % GPS) * LANES
                scat[j, pl.ds(k, LANES)] = sidx[j, pl.ds(k, LANES)]
            for j in range(NSUB):
                pltpu.async_copy(rows.at[pl.ds(j * SUB, SUB)],
                                 acc.at[scat.at[j]], ssems[b], add=True)

        def slot(b, nxt, has_next, has_prev):
            nb = 1 - b
            if has_next:
                pltpu.async_copy(meta_slice(nxt), metas[nb], msems[nb])
            drain_rows_bytes(gsems[b], b)
            if weighted:
                multiply(b)
            if has_next:
                pltpu.make_async_copy(meta_slice(0), metas[nb],
                                      msems[nb]).wait()
                build_idx(nb)
            if has_prev:
                drain_rows_bytes(ssems[nb], nb)
            if has_next:
                issue_gather(nb)
            issue_scatter(b)

        pltpu.sync_copy(meta_slice(0), meta0)
        build_idx(0)
        issue_gather(0)

        slot(0, 1, True, False)

        @pl.loop(1, NCHUNK - 2, step=2)
        def _(ci):
            slot(1, ci + 1, True, True)
            slot(0, ci + 2, True, True)

        slot(1, 0, False, True)
        drain_rows_bytes(ssems[1], 1)
        plsc.subcore_barrier()
        pltpu.sync_copy(
            acc.at[pl.ds(sid * STRIPE, STRIPE)],
            out_hbm.at[pl.ds(cid * ACC_ROWS + sid * STRIPE, STRIPE)])

    return seg


_seg_weighted = _make_seg(True)
_seg_unweighted = _make_seg(False)


def _make_cnt():
    mesh = plsc.VectorSubcoreMesh(core_axis_name="c", subcore_axis_name="s")
    scratch = [
        pltpu.VMEM((CNT_CH,), jnp.int32),
        pltpu.VMEM((CNT_CH,), jnp.int32),
        pltpu.VMEM((CNT_DOM,), jnp.float32),
        pltpu.VMEM((CNT_STR,), jnp.float32),
        pltpu.VMEM((CNT_STR,), jnp.float32),
        pltpu.SemaphoreType.DMA,
        pltpu.SemaphoreType.DMA,
        pltpu.VMEM_SHARED((NS, CNT_DOM), jnp.float32),
    ]

    @functools.partial(
        pl.kernel,
        out_type=jax.ShapeDtypeStruct((CNT_P,), jnp.float32),
        mesh=mesh,
        scratch_types=scratch,
        compiler_params=pltpu.CompilerParams(
            needs_layout_passes=False, use_tc_tiling_on_sc=False),
    )
    def cnt(dst_hbm, out_hbm, dstb0, dstb1, hist, tmp, tmp2,
            csem0, csem1, shared):
        cid = lax.axis_index("c")
        sid = lax.axis_index("s")
        base = cid * HALF
        tile_e0 = sid * (E // NS)
        dstbs = (dstb0, dstb1)
        csems = (csem0, csem1)

        @pl.loop(0, CNT_DOM // LANES)
        def _(r):
            hist[pl.ds(r * LANES, LANES)] = jnp.zeros((LANES,), jnp.float32)

        ones = jnp.ones((LANES,), jnp.float32)

        def issue_load(ci, b):
            pltpu.async_copy(dst_hbm.at[pl.ds(tile_e0 + ci * CNT_CH, CNT_CH)],
                             dstbs[b], csems[b])

        def wait_load(b):
            pltpu.make_async_copy(dst_hbm.at[pl.ds(0, CNT_CH)], dstbs[b],
                                  csems[b]).wait()

        def scan(b):
            dstb = dstbs[b]

            @pl.loop(0, CNT_CH // LANES)
            def _(g):
                dv = dstb[pl.ds(g * LANES, LANES)] - base
                ok = (dv >= 0) & (dv < HALF)
                adj = jnp.where(ok, dv, TRASH)
                plsc.addupdate_scatter(hist, [adj], ones)

        issue_load(0, 0)
        issue_load(1, 1)
        wait_load(0)
        scan(0)
        issue_load(2, 0)

        @pl.loop(1, CNT_NCH - 1, step=2)
        def _(ci):
            wait_load(1)
            scan(1)
            issue_load(ci + 2, 1)
            wait_load(0)
            scan(0)
            issue_load(ci + 3, 0)

        wait_load(1)
        wait_load(0)

        pltpu.sync_copy(hist, shared.at[sid])
        plsc.subcore_barrier()

        s0 = sid * CNT_STR
        pltpu.sync_copy(shared.at[0, pl.ds(s0, CNT_STR)], tmp)

        @pl.loop(1, NS)
        def _(t):
            pltpu.sync_copy(shared.at[t, pl.ds(s0, CNT_STR)], tmp2)

            @pl.loop(0, CNT_STR // LANES)
            def _(r):
                tmp[pl.ds(r * LANES, LANES)] = (
                    tmp[pl.ds(r * LANES, LANES)]
                    + tmp2[pl.ds(r * LANES, LANES)])

        @pl.loop(0, CNT_STR // LANES)
        def _(r):
            v = tmp[pl.ds(r * LANES, LANES)]
            tmp[pl.ds(r * LANES, LANES)] = 1.0 / jnp.maximum(v, 1.0)
        pltpu.sync_copy(tmp, out_hbm.at[pl.ds(cid * CNT_DOM + s0, CNT_STR)])

    return cnt


_cnt_kernel = _make_cnt()


_TC_BLK = 128
_TC_GRID = N_P // _TC_BLK


def _emb_body(x_ref, w_ref, b_ref, o_ref):
    o_ref[...] = jnp.maximum(
        jnp.dot(x_ref[...], w_ref[...], preferred_element_type=jnp.float32)
        + b_ref[...], 0.0)


def _emb_tc(x_p, w, b):
    return pl.pallas_call(
        _emb_body,
        grid=(_TC_GRID,),
        in_specs=[
            pl.BlockSpec((_TC_BLK, D_IN), lambda i: (i, 0)),
            pl.BlockSpec((D_IN, H), lambda i: (0, 0)),
            pl.BlockSpec((1, H), lambda i: (0, 0)),
        ],
        out_specs=pl.BlockSpec((_TC_BLK, H), lambda i: (i, 0)),
        out_shape=jax.ShapeDtypeStruct((N_P, H), jnp.float32),
    )(x_p, w, b)


def _layer_body(relu, s_ref, inv_ref, h_ref, wl_ref, bl_ref, wr_ref, o_ref):
    agg = s_ref[...] * inv_ref[...]
    o = (jnp.dot(agg, wl_ref[...], preferred_element_type=jnp.float32)
         + jnp.dot(h_ref[...], wr_ref[...], preferred_element_type=jnp.float32)
         + bl_ref[...])
    nrm = jnp.sqrt(jnp.sum(o * o, axis=-1, keepdims=True))
    o = o / jnp.maximum(nrm, 1e-12)
    if relu:
        o = jnp.maximum(o, 0.0)
    o_ref[...] = o


def _make_layer_tc(relu, width):
    def run(s_p, inv_p, h_p, wl, bl, wr):
        return pl.pallas_call(
            functools.partial(_layer_body, relu),
            grid=(_TC_GRID,),
            in_specs=[
                pl.BlockSpec((_TC_BLK, H), lambda i: (i, 0)),
                pl.BlockSpec((_TC_BLK, 1), lambda i: (i, 0)),
                pl.BlockSpec((_TC_BLK, H), lambda i: (i, 0)),
                pl.BlockSpec((H, width), lambda i: (0, 0)),
                pl.BlockSpec((1, width), lambda i: (0, 0)),
                pl.BlockSpec((H, width), lambda i: (0, 0)),
            ],
            out_specs=pl.BlockSpec((_TC_BLK, width), lambda i: (i, 0)),
            out_shape=jax.ShapeDtypeStruct((N_P, width), jnp.float32),
        )(s_p, inv_p, h_p, wl, bl, wr)
    return run


_layer_hidden = _make_layer_tc(True, H)
_layer_out = _make_layer_tc(False, C)


def kernel(x, edge_index, edge_weight, W_emb, b_emb, Wl0, bl0, Wr0,
           Wl1, bl1, Wr1, Wl2, bl2, Wr2, Wlo, blo, Wro):
    src = edge_index[0]
    dst = edge_index[1]
    zpad = jnp.zeros((PAD, D_IN), jnp.float32)
    x_p = jnp.concatenate([x[:HALF], zpad, x[HALF:], zpad], axis=0)

    epad = E_P - E
    src_p = jnp.concatenate([src, jnp.zeros((epad,), jnp.int32)])
    dst_p = jnp.concatenate([dst, jnp.full((epad,), N, jnp.int32)])
    ew_p = jnp.concatenate([lax.bitcast_convert_type(edge_weight, jnp.int32),
                            jnp.zeros((epad,), jnp.int32)])
    meta = jnp.stack([src_p.reshape(MROWS, SUB), dst_p.reshape(MROWS, SUB),
                      ew_p.reshape(MROWS, SUB)], axis=1)

    h = _emb_tc(x_p, W_emb, b_emb.reshape(1, H))
    cnt_raw = _cnt_kernel(dst_p)
    inv_p = jnp.concatenate(
        [cnt_raw[:ACC_ROWS],
         cnt_raw[CNT_DOM:CNT_DOM + ACC_ROWS]]).reshape(N_P, 1)

    for Wl, bl, Wr in ((Wl0, bl0, Wr0), (Wl1, bl1, Wr1), (Wl2, bl2, Wr2)):
        s = _seg_weighted(h, meta)
        h = _layer_hidden(s, inv_p, h, Wl, bl.reshape(1, H), Wr)

    s = _seg_unweighted(h, meta)
    o_p = _layer_out(s, inv_p, h, Wlo, blo.reshape(1, C), Wro)
    return jnp.concatenate([o_p[:HALF], o_p[ACC_ROWS:ACC_ROWS + HALF]], axis=0)
